# Initial kernel scaffold; baseline (speedup 1.0000x reference)
import functools

import jax
import jax.numpy as jnp
from jax import lax
from jax.experimental import pallas as pl
from jax.experimental.pallas import tpu as pltpu
from jax.experimental.pallas import tpu_sc as plsc

B, P, FD, NVOX = 4, 25000, 256, 80000


_BV = 3200


def _matvec_body(x_ref, w_ref, b_ref, o_ref):
    w = w_ref[0, :]
    o_ref[:] = jnp.sum(x_ref[:] * w[None, :], axis=1) + b_ref[0, 0]


def _voxel_scores(feats, W, b2d):
    grid = NVOX // _BV
    return pl.pallas_call(
        _matvec_body,
        grid=(grid,),
        in_specs=[
            pl.BlockSpec((_BV, FD), lambda i: (i, 0)),
            pl.BlockSpec((1, FD), lambda i: (0, 0)),
            pl.BlockSpec((1, 1), lambda i: (0, 0)),
        ],
        out_specs=pl.BlockSpec((_BV,), lambda i: (i,)),
        out_shape=jax.ShapeDtypeStruct((NVOX,), jnp.float32),
    )(feats, W, b2d)


_NW = 32
_CH = 3200
_NPAD = _NW * _CH


def _make_sc_gather():
    mesh = plsc.VectorSubcoreMesh(core_axis_name="c", subcore_axis_name="s")

    @functools.partial(
        pl.kernel,
        mesh=mesh,
        out_type=jax.ShapeDtypeStruct((_NPAD,), jnp.float32),
        scratch_types=[
            pltpu.VMEM((_CH,), jnp.int32),
            pltpu.VMEM((_CH,), jnp.float32),
            pltpu.SemaphoreType.DMA,
        ],
    )
    def gather_k(scores_hbm, idx_hbm, out_hbm, idx_v, vals_v, sem):
        wid = lax.axis_index("s") * 2 + lax.axis_index("c")
        base = wid * _CH
        pltpu.sync_copy(idx_hbm.at[pl.ds(base, _CH)], idx_v)
        pltpu.async_copy(scores_hbm.at[idx_v], vals_v, sem).wait()
        pltpu.sync_copy(vals_v, out_hbm.at[pl.ds(base, _CH)])

    return gather_k


_sc_gather = _make_sc_gather()


def kernel(feats_backbone, quantize2original, W, b):
    scores = _voxel_scores(feats_backbone, W, b.reshape(1, 1))
    idx_pad = jnp.concatenate(
        [quantize2original,
         jnp.zeros((_NPAD - B * P,), dtype=jnp.int32)])
    gathered = _sc_gather(scores, idx_pad)
    return gathered[: B * P].reshape(B, P)


# speedup vs baseline: 5.6358x; 5.6358x over previous
import functools

import jax
import jax.numpy as jnp
from jax import lax
from jax.experimental import pallas as pl
from jax.experimental.pallas import tpu as pltpu
from jax.experimental.pallas import tpu_sc as plsc

B, P, FD, NVOX = 4, 25000, 256, 80000


_BV = 3200


def _matvec_body(x_ref, w_ref, b_ref, o_ref):
    i = pl.program_id(0)
    w = w_ref[0, :]
    o_ref[pl.ds(i * _BV, _BV)] = (
        jnp.sum(x_ref[:] * w[None, :], axis=1) + b_ref[0, 0])


def _voxel_scores(feats, W, b2d):
    grid = NVOX // _BV
    return pl.pallas_call(
        _matvec_body,
        grid=(grid,),
        in_specs=[
            pl.BlockSpec((_BV, FD), lambda i: (i, 0)),
            pl.BlockSpec((1, FD), lambda i: (0, 0)),
            pl.BlockSpec((1, 1), lambda i: (0, 0)),
        ],
        out_specs=pl.BlockSpec((NVOX,), lambda i: (0,)),
        out_shape=jax.ShapeDtypeStruct((NVOX,), jnp.float32),
    )(feats, W, b2d)


_NW = 32
_CH = 3200
_NPAD = _NW * _CH


@functools.lru_cache(maxsize=1)
def _make_sc_gather():
    mesh = plsc.VectorSubcoreMesh(core_axis_name="c", subcore_axis_name="s")

    @functools.partial(
        pl.kernel,
        mesh=mesh,
        out_type=jax.ShapeDtypeStruct((_NPAD,), jnp.float32),
        scratch_types=[
            pltpu.VMEM((_CH,), jnp.int32),
            pltpu.VMEM((_CH,), jnp.float32),
            pltpu.SemaphoreType.DMA,
        ],
    )
    def gather_k(scores_hbm, idx_hbm, out_hbm, idx_v, vals_v, sem):
        wid = lax.axis_index("s") * 2 + lax.axis_index("c")
        base = wid * _CH
        pltpu.sync_copy(idx_hbm.at[pl.ds(base, _CH)], idx_v)
        pltpu.async_copy(scores_hbm.at[idx_v], vals_v, sem).wait()
        pltpu.sync_copy(vals_v, out_hbm.at[pl.ds(base, _CH)])

    return gather_k


def kernel(feats_backbone, quantize2original, W, b):
    scores = _voxel_scores(feats_backbone, W, b.reshape(1, 1))
    idx_pad = jnp.concatenate(
        [quantize2original,
         jnp.zeros((_NPAD - B * P,), dtype=jnp.int32)])
    gathered = _make_sc_gather()(scores, idx_pad)
    return gathered[: B * P].reshape(B, P)
